# baseline (speedup 1.0000x reference)
import functools

import jax
import jax.numpy as jnp
from jax import lax
from jax.experimental import pallas as pl
from jax.experimental.pallas import tpu as pltpu
from jax.experimental.pallas import tpu_sc as plsc

N = 10000
E = 320000
D = 128
NS = 100
B = 100
LAT = 5

NUM_CORES = 2
NUM_SUBCORES = 16
NUM_WORKERS = NUM_CORES * NUM_SUBCORES

CHUNK = 128
TOT_CHUNKS = E // CHUNK
CPT_MAIN = TOT_CHUNKS // NUM_WORKERS
CPT_LAST = TOT_CHUNKS - 31 * CPT_MAIN

ZBLK = 200
NZB = N // ZBLK
ZB_ITERS = -(-NZB // NUM_SUBCORES)


def _sc_agg_body(x_hbm, z_hbm, src_hbm, dst_hbm, out_hbm,
                 srcbuf, dstbuf, rows, acc,
                 gsem0, gsem1, isem0, isem1, dsem0, dsem1):
    cid = lax.axis_index("c")
    sid = lax.axis_index("s")
    wid = cid * NUM_SUBCORES + sid

    nchunks = jnp.where(wid == NUM_WORKERS - 1, CPT_LAST, CPT_MAIN)
    cbase = wid * CPT_MAIN

    def idx_copy(c, slot, isem, dsem):
        off = (cbase + c) * CHUNK
        pltpu.async_copy(src_hbm.at[pl.ds(off, CHUNK)], srcbuf.at[slot],
                         isem)
        pltpu.async_copy(dst_hbm.at[pl.ds(off, CHUNK)], dstbuf.at[slot],
                         dsem)

    def idx_wait(slot, sem, buf):
        pltpu.make_async_copy(src_hbm.at[pl.ds(0, CHUNK)], buf.at[slot],
                              sem).wait()

    idx_copy(0, 0, isem0, dsem0)
    idx_copy(1, 1, isem1, dsem1)

    def zblock(k, _):
        blk = k * NUM_SUBCORES + sid

        @pl.when(blk < NZB)
        def _():
            pltpu.sync_copy(z_hbm, acc.at[pl.ds(blk * ZBLK, ZBLK)])
        return 0
    lax.fori_loop(0, ZB_ITERS, zblock, 0)
    plsc.subcore_barrier()

    rows0 = rows.at[0]
    rows1 = rows.at[1]

    def gather(slot, sem):
        pltpu.async_copy(x_hbm.at[srcbuf.at[slot]], rows.at[slot], sem)

    def gather_wait(slot, sem):
        pltpu.make_async_copy(x_hbm.at[srcbuf.at[slot]], rows.at[slot],
                              sem).wait()

    idx_wait(0, isem0, srcbuf)
    gather(0, gsem0)

    def pair_body(i, _):
        j0 = 2 * i
        j1 = 2 * i + 1
        idx_wait(1, isem1, srcbuf)
        gather(1, gsem1)
        idx_wait(0, dsem0, dstbuf)
        gather_wait(0, gsem0)
        pltpu.sync_copy(rows0, acc.at[dstbuf.at[0]], add=True)

        @pl.when(j0 + 2 < nchunks)
        def _():
            idx_copy(j0 + 2, 0, isem0, dsem0)
        idx_wait(1, dsem1, dstbuf)
        gather_wait(1, gsem1)
        pltpu.sync_copy(rows1, acc.at[dstbuf.at[1]], add=True)

        @pl.when(j1 + 2 < nchunks)
        def _():
            idx_copy(j1 + 2, 1, isem1, dsem1)

        @pl.when(j0 + 2 < nchunks)
        def _():
            idx_wait(0, isem0, srcbuf)
            gather(0, gsem0)
        return 0
    lax.fori_loop(0, nchunks // 2, pair_body, 0)

    plsc.subcore_barrier()

    def wblock(k, _):
        blk = k * NUM_SUBCORES + sid

        @pl.when(blk < NZB)
        def _():
            pltpu.sync_copy(acc.at[pl.ds(blk * ZBLK, ZBLK)],
                            out_hbm.at[cid, pl.ds(blk * ZBLK, ZBLK)])
        return 0
    lax.fori_loop(0, ZB_ITERS, wblock, 0)


@functools.cache
def _sc_agg():
    return pl.kernel(
        _sc_agg_body,
        mesh=plsc.VectorSubcoreMesh(
            core_axis_name="c", subcore_axis_name="s",
            num_cores=NUM_CORES, num_subcores=NUM_SUBCORES),
        out_type=jax.ShapeDtypeStruct((NUM_CORES, N, D), jnp.float32),
        scratch_types=[
            pltpu.VMEM((2, CHUNK), jnp.int32),
            pltpu.VMEM((2, CHUNK), jnp.int32),
            pltpu.VMEM((2, CHUNK, D), jnp.float32),
            pltpu.VMEM_SHARED((N, D), jnp.float32),
            pltpu.SemaphoreType.DMA,
            pltpu.SemaphoreType.DMA,
            pltpu.SemaphoreType.DMA,
            pltpu.SemaphoreType.DMA,
            pltpu.SemaphoreType.DMA,
            pltpu.SemaphoreType.DMA,
        ],
    )


ROWS_TC = 1000


def _tc1_body(p_ref, wg_ref, wo_ref, wm_ref, uw_ref, lc_ref, u_ref):
    agg = p_ref[0] + p_ref[1]
    h = jnp.maximum(jnp.dot(agg, wg_ref[...],
                            preferred_element_type=jnp.float32), 0.0)
    od = jnp.dot(h, wo_ref[...], preferred_element_type=jnp.float32)
    q = jnp.dot(od, wm_ref[...], preferred_element_type=jnp.float32)
    col = lax.broadcasted_iota(jnp.int32, (ROWS_TC, NS), 1)
    rloc = lax.broadcasted_iota(jnp.int32, (ROWS_TC, NS), 0) % NS
    msk = col == rloc
    lc_ref[...] = jnp.concatenate(
        [jnp.sum(jnp.where(msk, q[:, l * NS:(l + 1) * NS], 0.0), axis=1,
                 keepdims=True) for l in range(LAT)], axis=1)
    u = (jnp.sum(h, axis=1) * (uw_ref[0] / D)
         + jnp.sum(agg, axis=1) * (uw_ref[1] / D))
    u_ref[...] = u.reshape(ROWS_TC, 1)


def _tc1(partials, W_gnn, W_od, W_mat, utility_w):
    grid = N // ROWS_TC
    return pl.pallas_call(
        _tc1_body,
        grid=(grid,),
        in_specs=[
            pl.BlockSpec((NUM_CORES, ROWS_TC, D), lambda i: (0, i, 0)),
            pl.BlockSpec((D, D), lambda i: (0, 0)),
            pl.BlockSpec((D, NS), lambda i: (0, 0)),
            pl.BlockSpec((NS, NS * LAT), lambda i: (0, 0)),
            pl.BlockSpec(memory_space=pltpu.SMEM),
        ],
        out_specs=[
            pl.BlockSpec((ROWS_TC, LAT), lambda i: (i, 0)),
            pl.BlockSpec((ROWS_TC, 1), lambda i: (i, 0)),
        ],
        out_shape=[
            jax.ShapeDtypeStruct((N, LAT), jnp.float32),
            jax.ShapeDtypeStruct((N, 1), jnp.float32),
        ],
    )(partials, W_gnn, W_od, W_mat, utility_w)


def _tc2_body(lc_ref, u_ref, be_ref, wt_ref, wb_ref, bb_ref, out_ref):
    grp = lax.broadcasted_iota(jnp.int32, (B, N), 0)
    row = lax.broadcasted_iota(jnp.int32, (B, N), 1)
    gmat = jnp.where(row // NS == grp, 1.0, 0.0)
    lat = jnp.maximum(
        jnp.dot(gmat, lc_ref[...], preferred_element_type=jnp.float32)
        + be_ref[...], 0.0)
    seat = lax.broadcasted_iota(jnp.int32, (N, NS), 1)
    rloc = lax.broadcasted_iota(jnp.int32, (N, NS), 0) % NS
    usel = jnp.where(seat == rloc, u_ref[...], 0.0)
    u2 = jnp.dot(gmat, usel, preferred_element_type=jnp.float32)
    m = jnp.max(u2, axis=1, keepdims=True)
    e = jnp.exp(u2 - m)
    prob = e / jnp.sum(e, axis=1, keepdims=True)
    out_ref[...] = (jnp.dot(prob, wt_ref[...],
                            preferred_element_type=jnp.float32)
                    + jnp.dot(lat, wb_ref[...],
                              preferred_element_type=jnp.float32)
                    + bb_ref[...])


def _tc2(lc, u, b_enc, W_top, W_bot, bias_big):
    return pl.pallas_call(
        _tc2_body,
        out_shape=jax.ShapeDtypeStruct((B, NS + N), jnp.float32),
    )(lc, u, b_enc.reshape(1, LAT), W_top, W_bot, bias_big)


def kernel(x_od, edge_index, W_gnn, W_od, W_enc, b_enc, W_dec, b_dec, utility_w):
    zsrc = jnp.zeros((ZBLK, D), jnp.float32)
    partials = _sc_agg()(x_od, zsrc, edge_index[0], edge_index[1])
    W_mat = W_enc.reshape(NS, NS, LAT).transpose(1, 2, 0).reshape(NS, NS * LAT)
    W_top = jnp.concatenate(
        [jnp.eye(NS, dtype=jnp.float32), jnp.zeros((NS, N), jnp.float32)],
        axis=1)
    W_bot = jnp.concatenate(
        [jnp.zeros((LAT, NS), jnp.float32), W_dec], axis=1)
    bias_big = jnp.concatenate(
        [jnp.zeros((NS,), jnp.float32), b_dec]).reshape(1, NS + N)
    lc, u = _tc1(partials, W_gnn, W_od, W_mat, utility_w)
    return _tc2(lc, u, b_enc, W_top, W_bot, bias_big)


# speedup vs baseline: 1.1082x; 1.1082x over previous
import functools

import jax
import jax.numpy as jnp
from jax import lax
from jax.experimental import pallas as pl
from jax.experimental.pallas import tpu as pltpu
from jax.experimental.pallas import tpu_sc as plsc

N = 10000
E = 320000
D = 128
NS = 100
B = 100
LAT = 5

NUM_CORES = 2
NUM_SUBCORES = 16
NUM_WORKERS = NUM_CORES * NUM_SUBCORES

CHUNK = 128
TOT_CHUNKS = E // CHUNK
CPT = 80
MAIN_CH = 16 * CPT
IDX_BLK = 16
TAIL = TOT_CHUNKS - (TOT_CHUNKS // IDX_BLK) * IDX_BLK
TAIL_BASE = TOT_CHUNKS - TAIL

ZBLK = 200
NZB = N // ZBLK
ZB_ITERS = -(-NZB // NUM_SUBCORES)


def _sc_agg_body(x_hbm, z_hbm, src_hbm, dst_hbm, out_hbm,
                 srcblk, dstblk, rows, srctail, dsttail, acc, sem0, sem1):
    cid = lax.axis_index("c")
    sid = lax.axis_index("s")
    wid = cid * NUM_SUBCORES + sid

    cbase = wid * CPT
    nstages = jnp.where(wid == NUM_WORKERS - 1, 1, CPT // IDX_BLK)

    def stage_idx(k):
        off = cbase + k * IDX_BLK
        pltpu.sync_copy(src_hbm.at[pl.ds(off * CHUNK, IDX_BLK * CHUNK)],
                        srcblk.at[k % 2])
        pltpu.sync_copy(dst_hbm.at[pl.ds(off, IDX_BLK)], dstblk.at[k % 2])

    stage_idx(0)

    def zblock(k, _):
        blk = k * NUM_SUBCORES + sid

        @pl.when(blk < NZB)
        def _():
            pltpu.sync_copy(z_hbm, acc.at[pl.ds(blk * ZBLK, ZBLK)])
        return 0
    lax.fori_loop(0, ZB_ITERS, zblock, 0)
    plsc.subcore_barrier()

    rows0 = rows.at[0]
    rows1 = rows.at[1]

    def stage_body(k, _):
        sb = srcblk.at[k % 2]
        db = dstblk.at[k % 2]
        pltpu.async_copy(x_hbm.at[sb.at[pl.ds(0, CHUNK)]], rows0, sem0)

        @pl.when(k + 1 < nstages)
        def _():
            stage_idx(k + 1)

        def pair_body(j, _):
            c0 = 2 * j
            c1 = 2 * j + 1
            pltpu.async_copy(x_hbm.at[sb.at[pl.ds(c1 * CHUNK, CHUNK)]],
                             rows1, sem1)
            pltpu.make_async_copy(
                x_hbm.at[sb.at[pl.ds(c0 * CHUNK, CHUNK)]], rows0, sem0).wait()
            pltpu.sync_copy(rows0, acc.at[db.at[c0]], add=True)

            @pl.when(c1 + 1 < IDX_BLK)
            def _():
                pltpu.async_copy(
                    x_hbm.at[sb.at[pl.ds(jnp.minimum(c1 + 1, IDX_BLK - 1)
                                         * CHUNK, CHUNK)]], rows0, sem0)
            pltpu.make_async_copy(
                x_hbm.at[sb.at[pl.ds(c1 * CHUNK, CHUNK)]], rows1, sem1).wait()
            pltpu.sync_copy(rows1, acc.at[db.at[c1]], add=True)
            return 0
        lax.fori_loop(0, IDX_BLK // 2, pair_body, 0)
        return 0
    lax.fori_loop(0, nstages, stage_body, 0)

    @pl.when(wid == NUM_WORKERS - 1)
    def _():
        pltpu.sync_copy(src_hbm.at[pl.ds(TAIL_BASE * CHUNK, TAIL * CHUNK)],
                        srctail)
        pltpu.sync_copy(dst_hbm.at[pl.ds(TAIL_BASE, TAIL)], dsttail)
        for t in range(TAIL):
            pltpu.async_copy(
                x_hbm.at[srctail.at[pl.ds(t * CHUNK, CHUNK)]],
                rows0, sem0).wait()
            pltpu.sync_copy(rows0, acc.at[dsttail.at[t]], add=True)

    plsc.subcore_barrier()

    def wblock(k, _):
        blk = k * NUM_SUBCORES + sid

        @pl.when(blk < NZB)
        def _():
            pltpu.sync_copy(acc.at[pl.ds(blk * ZBLK, ZBLK)],
                            out_hbm.at[cid, pl.ds(blk * ZBLK, ZBLK)])
        return 0
    lax.fori_loop(0, ZB_ITERS, wblock, 0)


@functools.cache
def _sc_agg():
    return pl.kernel(
        _sc_agg_body,
        mesh=plsc.VectorSubcoreMesh(
            core_axis_name="c", subcore_axis_name="s",
            num_cores=NUM_CORES, num_subcores=NUM_SUBCORES),
        out_type=jax.ShapeDtypeStruct((NUM_CORES, N, D), jnp.float32),
        scratch_types=[
            pltpu.VMEM((2, IDX_BLK * CHUNK), jnp.int32),
            pltpu.VMEM((2, IDX_BLK, CHUNK), jnp.int32),
            pltpu.VMEM((2, CHUNK, D), jnp.float32),
            pltpu.VMEM((TAIL * CHUNK,), jnp.int32),
            pltpu.VMEM((TAIL, CHUNK), jnp.int32),
            pltpu.VMEM_SHARED((N, D), jnp.float32),
            pltpu.SemaphoreType.DMA,
            pltpu.SemaphoreType.DMA,
        ],
    )


ROWS_TC = 1000


def _tc1_body(p_ref, wg_ref, wo_ref, wm_ref, uw_ref, lc_ref, u_ref):
    agg = p_ref[0] + p_ref[1]
    h = jnp.maximum(jnp.dot(agg, wg_ref[...],
                            preferred_element_type=jnp.float32), 0.0)
    od = jnp.dot(h, wo_ref[...], preferred_element_type=jnp.float32)
    q = jnp.dot(od, wm_ref[...], preferred_element_type=jnp.float32)
    col = lax.broadcasted_iota(jnp.int32, (ROWS_TC, NS), 1)
    rloc = lax.broadcasted_iota(jnp.int32, (ROWS_TC, NS), 0) % NS
    msk = col == rloc
    lc_ref[...] = jnp.concatenate(
        [jnp.sum(jnp.where(msk, q[:, l * NS:(l + 1) * NS], 0.0), axis=1,
                 keepdims=True) for l in range(LAT)], axis=1)
    u = (jnp.sum(h, axis=1) * (uw_ref[0] / D)
         + jnp.sum(agg, axis=1) * (uw_ref[1] / D))
    u_ref[...] = u.reshape(ROWS_TC, 1)


def _tc1(partials, W_gnn, W_od, W_mat, utility_w):
    grid = N // ROWS_TC
    return pl.pallas_call(
        _tc1_body,
        grid=(grid,),
        in_specs=[
            pl.BlockSpec((NUM_CORES, ROWS_TC, D), lambda i: (0, i, 0)),
            pl.BlockSpec((D, D), lambda i: (0, 0)),
            pl.BlockSpec((D, NS), lambda i: (0, 0)),
            pl.BlockSpec((NS, NS * LAT), lambda i: (0, 0)),
            pl.BlockSpec(memory_space=pltpu.SMEM),
        ],
        out_specs=[
            pl.BlockSpec((ROWS_TC, LAT), lambda i: (i, 0)),
            pl.BlockSpec((ROWS_TC, 1), lambda i: (i, 0)),
        ],
        out_shape=[
            jax.ShapeDtypeStruct((N, LAT), jnp.float32),
            jax.ShapeDtypeStruct((N, 1), jnp.float32),
        ],
    )(partials, W_gnn, W_od, W_mat, utility_w)


def _tc2_body(lc_ref, u_ref, be_ref, wt_ref, wb_ref, bb_ref, out_ref):
    grp = lax.broadcasted_iota(jnp.int32, (B, N), 0)
    row = lax.broadcasted_iota(jnp.int32, (B, N), 1)
    gmat = jnp.where(row // NS == grp, 1.0, 0.0)
    lat = jnp.maximum(
        jnp.dot(gmat, lc_ref[...], preferred_element_type=jnp.float32)
        + be_ref[...], 0.0)
    seat = lax.broadcasted_iota(jnp.int32, (N, NS), 1)
    rloc = lax.broadcasted_iota(jnp.int32, (N, NS), 0) % NS
    usel = jnp.where(seat == rloc, u_ref[...], 0.0)
    u2 = jnp.dot(gmat, usel, preferred_element_type=jnp.float32)
    m = jnp.max(u2, axis=1, keepdims=True)
    e = jnp.exp(u2 - m)
    prob = e / jnp.sum(e, axis=1, keepdims=True)
    out_ref[...] = (jnp.dot(prob, wt_ref[...],
                            preferred_element_type=jnp.float32)
                    + jnp.dot(lat, wb_ref[...],
                              preferred_element_type=jnp.float32)
                    + bb_ref[...])


def _tc2(lc, u, b_enc, W_top, W_bot, bias_big):
    return pl.pallas_call(
        _tc2_body,
        out_shape=jax.ShapeDtypeStruct((B, NS + N), jnp.float32),
    )(lc, u, b_enc.reshape(1, LAT), W_top, W_bot, bias_big)


def kernel(x_od, edge_index, W_gnn, W_od, W_enc, b_enc, W_dec, b_dec, utility_w):
    zsrc = jnp.zeros((ZBLK, D), jnp.float32)
    dst2d = edge_index[1].reshape(TOT_CHUNKS, CHUNK)
    partials = _sc_agg()(x_od, zsrc, edge_index[0], dst2d)
    W_mat = W_enc.reshape(NS, NS, LAT).transpose(1, 2, 0).reshape(NS, NS * LAT)
    W_top = jnp.concatenate(
        [jnp.eye(NS, dtype=jnp.float32), jnp.zeros((NS, N), jnp.float32)],
        axis=1)
    W_bot = jnp.concatenate(
        [jnp.zeros((LAT, NS), jnp.float32), W_dec], axis=1)
    bias_big = jnp.concatenate(
        [jnp.zeros((NS,), jnp.float32), b_dec]).reshape(1, NS + N)
    lc, u = _tc1(partials, W_gnn, W_od, W_mat, utility_w)
    return _tc2(lc, u, b_enc, W_top, W_bot, bias_big)


# speedup vs baseline: 1.3519x; 1.2198x over previous
import functools

import jax
import jax.numpy as jnp
from jax import lax
from jax.experimental import pallas as pl
from jax.experimental.pallas import tpu as pltpu
from jax.experimental.pallas import tpu_sc as plsc

N = 10000
E = 320000
D = 128
NS = 100
B = 100
LAT = 5

NUM_CORES = 2
NUM_SUBCORES = 16
NUM_WORKERS = NUM_CORES * NUM_SUBCORES

CHUNK = 128
TOT_CHUNKS = E // CHUNK
CPT = 78
IDX_BLK = 26
NSTAGES = CPT // IDX_BLK
TAIL = TOT_CHUNKS - NUM_WORKERS * CPT
TAIL_BASE = NUM_WORKERS * CPT
LANES = 16

ZBLK = 200
NZB = N // ZBLK
ZB_ITERS = -(-NZB // NUM_SUBCORES)


def _sc_agg_body(x_hbm, z_hbm, src_hbm, dst_hbm, out_hbm,
                 srcblk, dst1d, dstblk, rows, acc, sem0, sem1):
    cid = lax.axis_index("c")
    sid = lax.axis_index("s")
    wid = cid * NUM_SUBCORES + sid

    cbase = wid * CPT

    def stage_idx(k):
        off = (cbase + k * IDX_BLK) * CHUNK
        pltpu.sync_copy(src_hbm.at[pl.ds(off, IDX_BLK * CHUNK)],
                        srcblk.at[k % 2])
        pltpu.sync_copy(dst_hbm.at[pl.ds(off, IDX_BLK * CHUNK)], dst1d)

        def rcopy(t, _):
            c = t // (CHUNK // LANES)
            b = (t % (CHUNK // LANES)) * LANES
            dstblk[k % 2, c, pl.ds(b, LANES)] = dst1d[pl.ds(c * CHUNK + b,
                                                            LANES)]
            return 0
        lax.fori_loop(0, IDX_BLK * (CHUNK // LANES), rcopy, 0)

    stage_idx(0)

    def zblock(k, _):
        blk = k * NUM_SUBCORES + sid

        @pl.when(blk < NZB)
        def _():
            pltpu.sync_copy(z_hbm, acc.at[pl.ds(blk * ZBLK, ZBLK)])
        return 0
    lax.fori_loop(0, ZB_ITERS, zblock, 0)
    plsc.subcore_barrier()

    rows0 = rows.at[0]
    rows1 = rows.at[1]
    for k in range(NSTAGES):
        sb = srcblk.at[k % 2]
        db = dstblk.at[k % 2]
        pltpu.async_copy(x_hbm.at[sb.at[pl.ds(0, CHUNK)]], rows0, sem0)
        if k + 1 < NSTAGES:
            stage_idx(k + 1)

        def pair_body(j, _):
            c0 = 2 * j
            c1 = 2 * j + 1
            pltpu.async_copy(x_hbm.at[sb.at[pl.ds(c1 * CHUNK, CHUNK)]],
                             rows1, sem1)
            pltpu.make_async_copy(
                x_hbm.at[sb.at[pl.ds(c0 * CHUNK, CHUNK)]], rows0, sem0).wait()
            pltpu.sync_copy(rows0, acc.at[db.at[c0]], add=True)

            @pl.when(c1 + 1 < IDX_BLK)
            def _():
                pltpu.async_copy(
                    x_hbm.at[sb.at[pl.ds(jnp.minimum(c1 + 1, IDX_BLK - 1)
                                         * CHUNK, CHUNK)]], rows0, sem0)
            pltpu.make_async_copy(
                x_hbm.at[sb.at[pl.ds(c1 * CHUNK, CHUNK)]], rows1, sem1).wait()
            pltpu.sync_copy(rows1, acc.at[db.at[c1]], add=True)
            return 0
        lax.fori_loop(0, IDX_BLK // 2, pair_body, 0)

    @pl.when(wid < TAIL)
    def _():
        off = (TAIL_BASE + wid) * CHUNK
        pltpu.sync_copy(src_hbm.at[pl.ds(off, CHUNK)],
                        srcblk.at[0, pl.ds(0, CHUNK)])
        pltpu.sync_copy(dst_hbm.at[pl.ds(off, CHUNK)],
                        dst1d.at[pl.ds(0, CHUNK)])

        def rcopy(t, _):
            dstblk[0, 0, pl.ds(t * LANES, LANES)] = dst1d[pl.ds(t * LANES,
                                                               LANES)]
            return 0
        lax.fori_loop(0, CHUNK // LANES, rcopy, 0)
        pltpu.async_copy(x_hbm.at[srcblk.at[0, pl.ds(0, CHUNK)]],
                         rows0, sem0).wait()
        pltpu.sync_copy(rows0, acc.at[dstblk.at[0, 0]], add=True)

    plsc.subcore_barrier()

    def wblock(k, _):
        blk = k * NUM_SUBCORES + sid

        @pl.when(blk < NZB)
        def _():
            pltpu.sync_copy(acc.at[pl.ds(blk * ZBLK, ZBLK)],
                            out_hbm.at[cid, pl.ds(blk * ZBLK, ZBLK)])
        return 0
    lax.fori_loop(0, ZB_ITERS, wblock, 0)


@functools.cache
def _sc_agg():
    return pl.kernel(
        _sc_agg_body,
        mesh=plsc.VectorSubcoreMesh(
            core_axis_name="c", subcore_axis_name="s",
            num_cores=NUM_CORES, num_subcores=NUM_SUBCORES),
        out_type=jax.ShapeDtypeStruct((NUM_CORES, N, D), jnp.float32),
        scratch_types=[
            pltpu.VMEM((2, IDX_BLK * CHUNK), jnp.int32),
            pltpu.VMEM((IDX_BLK * CHUNK,), jnp.int32),
            pltpu.VMEM((2, IDX_BLK, CHUNK), jnp.int32),
            pltpu.VMEM((2, CHUNK, D), jnp.float32),
            pltpu.VMEM_SHARED((N, D), jnp.float32),
            pltpu.SemaphoreType.DMA,
            pltpu.SemaphoreType.DMA,
        ],
    )


ROWS_TC = 1000


def _tc1_body(p_ref, wg_ref, wo_ref, wm_ref, uw_ref, lc_ref, u_ref):
    agg = p_ref[0] + p_ref[1]
    h = jnp.maximum(jnp.dot(agg, wg_ref[...],
                            preferred_element_type=jnp.float32), 0.0)
    od = jnp.dot(h, wo_ref[...], preferred_element_type=jnp.float32)
    q = jnp.dot(od, wm_ref[...], preferred_element_type=jnp.float32)
    col = lax.broadcasted_iota(jnp.int32, (ROWS_TC, D), 1)
    rloc = lax.broadcasted_iota(jnp.int32, (ROWS_TC, D), 0) % NS
    msk = col == rloc
    lc_ref[...] = jnp.concatenate(
        [jnp.sum(jnp.where(msk, q[:, l * D:(l + 1) * D], 0.0), axis=1,
                 keepdims=True) for l in range(LAT)], axis=1)
    u = (jnp.sum(h, axis=1) * (uw_ref[0] / D)
         + jnp.sum(agg, axis=1) * (uw_ref[1] / D))
    u_ref[...] = u.reshape(ROWS_TC, 1)


def _tc1(partials, W_gnn, W_od, W_mat, utility_w):
    grid = N // ROWS_TC
    return pl.pallas_call(
        _tc1_body,
        grid=(grid,),
        in_specs=[
            pl.BlockSpec((NUM_CORES, ROWS_TC, D), lambda i: (0, i, 0)),
            pl.BlockSpec((D, D), lambda i: (0, 0)),
            pl.BlockSpec((D, NS), lambda i: (0, 0)),
            pl.BlockSpec((NS, D * LAT), lambda i: (0, 0)),
            pl.BlockSpec(memory_space=pltpu.SMEM),
        ],
        out_specs=[
            pl.BlockSpec((ROWS_TC, LAT), lambda i: (i, 0)),
            pl.BlockSpec((ROWS_TC, 1), lambda i: (i, 0)),
        ],
        out_shape=[
            jax.ShapeDtypeStruct((N, LAT), jnp.float32),
            jax.ShapeDtypeStruct((N, 1), jnp.float32),
        ],
    )(partials, W_gnn, W_od, W_mat, utility_w)


def _tc2_body(lc_ref, u_ref, be_ref, wt_ref, wb_ref, bb_ref, out_ref):
    grp = lax.broadcasted_iota(jnp.int32, (B, N), 0)
    row = lax.broadcasted_iota(jnp.int32, (B, N), 1)
    gmat = jnp.where(row // NS == grp, 1.0, 0.0)
    lat = jnp.maximum(
        jnp.dot(gmat, lc_ref[...], preferred_element_type=jnp.float32)
        + be_ref[...], 0.0)
    seat = lax.broadcasted_iota(jnp.int32, (N, NS), 1)
    rloc = lax.broadcasted_iota(jnp.int32, (N, NS), 0) % NS
    usel = jnp.where(seat == rloc, u_ref[...], 0.0)
    u2 = jnp.dot(gmat, usel, preferred_element_type=jnp.float32)
    m = jnp.max(u2, axis=1, keepdims=True)
    e = jnp.exp(u2 - m)
    prob = e / jnp.sum(e, axis=1, keepdims=True)
    out_ref[...] = (jnp.dot(prob, wt_ref[...],
                            preferred_element_type=jnp.float32)
                    + jnp.dot(lat, wb_ref[...],
                              preferred_element_type=jnp.float32)
                    + bb_ref[...])


def _tc2(lc, u, b_enc, W_top, W_bot, bias_big):
    return pl.pallas_call(
        _tc2_body,
        out_shape=jax.ShapeDtypeStruct((B, NS + N), jnp.float32),
    )(lc, u, b_enc.reshape(1, LAT), W_top, W_bot, bias_big)


def kernel(x_od, edge_index, W_gnn, W_od, W_enc, b_enc, W_dec, b_dec, utility_w):
    zsrc = jnp.zeros((ZBLK, D), jnp.float32)
    partials = _sc_agg()(x_od, zsrc, edge_index[0], edge_index[1])
    W_mat = jnp.pad(W_enc.reshape(NS, NS, LAT).transpose(1, 2, 0),
                    ((0, 0), (0, 0), (0, D - NS))).reshape(NS, D * LAT)
    W_top = jnp.concatenate(
        [jnp.eye(NS, dtype=jnp.float32), jnp.zeros((NS, N), jnp.float32)],
        axis=1)
    W_bot = jnp.concatenate(
        [jnp.zeros((LAT, NS), jnp.float32), W_dec], axis=1)
    bias_big = jnp.concatenate(
        [jnp.zeros((NS,), jnp.float32), b_dec]).reshape(1, NS + N)
    lc, u = _tc1(partials, W_gnn, W_od, W_mat, utility_w)
    return _tc2(lc, u, b_enc, W_top, W_bot, bias_big)


# speedup vs baseline: 1.3752x; 1.0173x over previous
import functools

import jax
import jax.numpy as jnp
from jax import lax
from jax.experimental import pallas as pl
from jax.experimental.pallas import tpu as pltpu
from jax.experimental.pallas import tpu_sc as plsc

N = 10000
E = 320000
D = 128
NS = 100
B = 100
LAT = 5

NUM_CORES = 2
NUM_SUBCORES = 16
NUM_WORKERS = NUM_CORES * NUM_SUBCORES

CHUNK = 128
TOT_CHUNKS = E // CHUNK
CPT = 78
IDX_BLK = 26
NSTAGES = CPT // IDX_BLK
TAIL = TOT_CHUNKS - NUM_WORKERS * CPT
TAIL_BASE = NUM_WORKERS * CPT
LANES = 16

ZBLK = 200
NZB = N // ZBLK
ZB_ITERS = -(-NZB // NUM_SUBCORES)


def _sc_agg_body(x_hbm, z_hbm, src_hbm, dst_hbm, out_hbm,
                 srcblk, dst1d, dstblk, rows, acc, sem0, sem1):
    cid = lax.axis_index("c")
    sid = lax.axis_index("s")
    wid = cid * NUM_SUBCORES + sid

    cbase = wid * CPT

    def stage_idx(k):
        off = (cbase + k * IDX_BLK) * CHUNK
        pltpu.sync_copy(src_hbm.at[pl.ds(off, IDX_BLK * CHUNK)],
                        srcblk.at[k % 2])
        pltpu.sync_copy(dst_hbm.at[pl.ds(off, IDX_BLK * CHUNK)], dst1d)

        def rcopy(t, _):
            c = t // (CHUNK // LANES)
            b = (t % (CHUNK // LANES)) * LANES
            dstblk[k % 2, c, pl.ds(b, LANES)] = dst1d[pl.ds(c * CHUNK + b,
                                                            LANES)]
            return 0
        lax.fori_loop(0, IDX_BLK * (CHUNK // LANES), rcopy, 0)

    stage_idx(0)

    def zblock(k, _):
        blk = k * NUM_SUBCORES + sid

        @pl.when(blk < NZB)
        def _():
            pltpu.sync_copy(z_hbm, acc.at[pl.ds(blk * ZBLK, ZBLK)])
        return 0
    lax.fori_loop(0, ZB_ITERS, zblock, 0)
    plsc.subcore_barrier()

    rows0 = rows.at[0]
    rows1 = rows.at[1]
    for k in range(NSTAGES):
        sb = srcblk.at[k % 2]
        db = dstblk.at[k % 2]
        pltpu.async_copy(x_hbm.at[sb.at[pl.ds(0, CHUNK)]], rows0, sem0)
        if k + 1 < NSTAGES:
            stage_idx(k + 1)

        def pair_body(j, _):
            c0 = 2 * j
            c1 = 2 * j + 1
            pltpu.async_copy(x_hbm.at[sb.at[pl.ds(c1 * CHUNK, CHUNK)]],
                             rows1, sem1)
            pltpu.make_async_copy(
                x_hbm.at[sb.at[pl.ds(c0 * CHUNK, CHUNK)]], rows0, sem0).wait()
            pltpu.sync_copy(rows0, acc.at[db.at[c0]], add=True)

            @pl.when(c1 + 1 < IDX_BLK)
            def _():
                pltpu.async_copy(
                    x_hbm.at[sb.at[pl.ds(jnp.minimum(c1 + 1, IDX_BLK - 1)
                                         * CHUNK, CHUNK)]], rows0, sem0)
            pltpu.make_async_copy(
                x_hbm.at[sb.at[pl.ds(c1 * CHUNK, CHUNK)]], rows1, sem1).wait()
            pltpu.sync_copy(rows1, acc.at[db.at[c1]], add=True)
            return 0
        lax.fori_loop(0, IDX_BLK // 2, pair_body, 0)

    @pl.when(wid < TAIL)
    def _():
        off = (TAIL_BASE + wid) * CHUNK
        pltpu.sync_copy(src_hbm.at[pl.ds(off, CHUNK)],
                        srcblk.at[0, pl.ds(0, CHUNK)])
        pltpu.sync_copy(dst_hbm.at[pl.ds(off, CHUNK)],
                        dst1d.at[pl.ds(0, CHUNK)])

        def rcopy(t, _):
            dstblk[0, 0, pl.ds(t * LANES, LANES)] = dst1d[pl.ds(t * LANES,
                                                               LANES)]
            return 0
        lax.fori_loop(0, CHUNK // LANES, rcopy, 0)
        pltpu.async_copy(x_hbm.at[srcblk.at[0, pl.ds(0, CHUNK)]],
                         rows0, sem0).wait()
        pltpu.sync_copy(rows0, acc.at[dstblk.at[0, 0]], add=True)

    plsc.subcore_barrier()

    def wblock(k, _):
        blk = k * NUM_SUBCORES + sid

        @pl.when(blk < NZB)
        def _():
            pltpu.sync_copy(acc.at[pl.ds(blk * ZBLK, ZBLK)],
                            out_hbm.at[cid, pl.ds(blk * ZBLK, ZBLK)])
        return 0
    lax.fori_loop(0, ZB_ITERS, wblock, 0)


@functools.cache
def _sc_agg():
    return pl.kernel(
        _sc_agg_body,
        mesh=plsc.VectorSubcoreMesh(
            core_axis_name="c", subcore_axis_name="s",
            num_cores=NUM_CORES, num_subcores=NUM_SUBCORES),
        out_type=jax.ShapeDtypeStruct((NUM_CORES, N, D), jnp.float32),
        scratch_types=[
            pltpu.VMEM((2, IDX_BLK * CHUNK), jnp.int32),
            pltpu.VMEM((IDX_BLK * CHUNK,), jnp.int32),
            pltpu.VMEM((2, IDX_BLK, CHUNK), jnp.int32),
            pltpu.VMEM((2, CHUNK, D), jnp.float32),
            pltpu.VMEM_SHARED((N, D), jnp.float32),
            pltpu.SemaphoreType.DMA,
            pltpu.SemaphoreType.DMA,
        ],
    )


ROWS_TC = 1000


def _tc1_body(p_ref, wg_ref, wo_ref, wm_ref, uw_ref, lc_ref, u_ref):
    agg = p_ref[0] + p_ref[1]
    h = jnp.maximum(jnp.dot(agg, wg_ref[...],
                            preferred_element_type=jnp.float32), 0.0)
    od = jnp.dot(h, wo_ref[...], preferred_element_type=jnp.float32)
    q = jnp.dot(od, wm_ref[...], preferred_element_type=jnp.float32)
    col = lax.broadcasted_iota(jnp.int32, (ROWS_TC, D), 1)
    rloc = lax.broadcasted_iota(jnp.int32, (ROWS_TC, D), 0) % NS
    msk = col == rloc
    lc_ref[...] = jnp.concatenate(
        [jnp.sum(jnp.where(msk, q[:, l * D:(l + 1) * D], 0.0), axis=1,
                 keepdims=True) for l in range(LAT)], axis=1)
    u = (jnp.sum(h, axis=1) * (uw_ref[0] / D)
         + jnp.sum(agg, axis=1) * (uw_ref[1] / D))
    u_ref[...] = u.reshape(ROWS_TC, 1)


def _tc1(partials, W_gnn, W_od, W_mat, utility_w):
    grid = N // ROWS_TC
    return pl.pallas_call(
        _tc1_body,
        grid=(grid,),
        in_specs=[
            pl.BlockSpec((NUM_CORES, ROWS_TC, D), lambda i: (0, i, 0)),
            pl.BlockSpec((D, D), lambda i: (0, 0)),
            pl.BlockSpec((D, NS), lambda i: (0, 0)),
            pl.BlockSpec((NS, D * LAT), lambda i: (0, 0)),
            pl.BlockSpec(memory_space=pltpu.SMEM),
        ],
        out_specs=[
            pl.BlockSpec((ROWS_TC, LAT), lambda i: (i, 0)),
            pl.BlockSpec((ROWS_TC, 1), lambda i: (i, 0)),
        ],
        out_shape=[
            jax.ShapeDtypeStruct((N, LAT), jnp.float32),
            jax.ShapeDtypeStruct((N, 1), jnp.float32),
        ],
    )(partials, W_gnn, W_od, W_mat, utility_w)


def _tc2_body(lc_ref, u_ref, be_ref, wb_ref, bb_ref, out_ref):
    grp = lax.broadcasted_iota(jnp.int32, (B, N), 0)
    row = lax.broadcasted_iota(jnp.int32, (B, N), 1)
    gmat = jnp.where(row // NS == grp, 1.0, 0.0)
    lat = jnp.maximum(
        jnp.dot(gmat, lc_ref[...], preferred_element_type=jnp.float32)
        + be_ref[...], 0.0)
    seat = lax.broadcasted_iota(jnp.int32, (N, NS), 1)
    rloc = lax.broadcasted_iota(jnp.int32, (N, NS), 0) % NS
    usel = jnp.where(seat == rloc, u_ref[...], 0.0)
    u2 = jnp.dot(gmat, usel, preferred_element_type=jnp.float32)
    m = jnp.max(u2, axis=1, keepdims=True)
    e = jnp.exp(u2 - m)
    prob = e / jnp.sum(e, axis=1, keepdims=True)
    out_ref[...] = (jnp.dot(lat, wb_ref[...],
                            preferred_element_type=jnp.float32)
                    + bb_ref[...])
    out_ref[:, :NS] = prob


def _tc2(lc, u, b_enc, W_bot, bias_big):
    return pl.pallas_call(
        _tc2_body,
        out_shape=jax.ShapeDtypeStruct((B, NS + N), jnp.float32),
    )(lc, u, b_enc.reshape(1, LAT), W_bot, bias_big)


def kernel(x_od, edge_index, W_gnn, W_od, W_enc, b_enc, W_dec, b_dec, utility_w):
    zsrc = jnp.zeros((ZBLK, D), jnp.float32)
    partials = _sc_agg()(x_od, zsrc, edge_index[0], edge_index[1])
    W_mat = jnp.pad(W_enc.reshape(NS, NS, LAT).transpose(1, 2, 0),
                    ((0, 0), (0, 0), (0, D - NS))).reshape(NS, D * LAT)
    W_bot = jnp.concatenate(
        [jnp.zeros((LAT, NS), jnp.float32), W_dec], axis=1)
    bias_big = jnp.concatenate(
        [jnp.zeros((NS,), jnp.float32), b_dec]).reshape(1, NS + N)
    lc, u = _tc1(partials, W_gnn, W_od, W_mat, utility_w)
    return _tc2(lc, u, b_enc, W_bot, bias_big)


# speedup vs baseline: 1.4761x; 1.0734x over previous
import functools

import jax
import jax.numpy as jnp
from jax import lax
from jax.experimental import pallas as pl
from jax.experimental.pallas import tpu as pltpu
from jax.experimental.pallas import tpu_sc as plsc

N = 10000
E = 320000
D = 128
NS = 100
B = 100
LAT = 5

NUM_CORES = 2
NUM_SUBCORES = 16
NUM_WORKERS = NUM_CORES * NUM_SUBCORES

CHUNK = 128
TOT_CHUNKS = E // CHUNK
CPT = 78
IDX_BLK = 26
NSTAGES = CPT // IDX_BLK
TAIL = TOT_CHUNKS - NUM_WORKERS * CPT
TAIL_BASE = NUM_WORKERS * CPT
LANES = 16

ZBLK = 200
NZB = N // ZBLK
ZB_ITERS = -(-NZB // NUM_SUBCORES)


def _sc_agg_body(x_hbm, z_hbm, src_hbm, dst_hbm, out_hbm,
                 srcblk, dst1d, dstblk, rows, acc, sem0, sem1):
    cid = lax.axis_index("c")
    sid = lax.axis_index("s")
    wid = cid * NUM_SUBCORES + sid

    cbase = wid * CPT

    def stage_idx(k):
        off = (cbase + k * IDX_BLK) * CHUNK
        pltpu.sync_copy(src_hbm.at[pl.ds(off, IDX_BLK * CHUNK)],
                        srcblk.at[k % 2])
        pltpu.sync_copy(dst_hbm.at[pl.ds(off, IDX_BLK * CHUNK)], dst1d)

        def rcopy(t, _):
            c = t // (CHUNK // LANES)
            b = (t % (CHUNK // LANES)) * LANES
            dstblk[k % 2, c, pl.ds(b, LANES)] = dst1d[pl.ds(c * CHUNK + b,
                                                            LANES)]
            return 0
        lax.fori_loop(0, IDX_BLK * (CHUNK // LANES), rcopy, 0)

    stage_idx(0)

    def zblock(k, _):
        blk = k * NUM_SUBCORES + sid

        @pl.when(blk < NZB)
        def _():
            pltpu.sync_copy(z_hbm, acc.at[pl.ds(blk * ZBLK, ZBLK)])
        return 0
    lax.fori_loop(0, ZB_ITERS, zblock, 0)
    plsc.subcore_barrier()

    rows0 = rows.at[0]
    rows1 = rows.at[1]
    for k in range(NSTAGES):
        sb = srcblk.at[k % 2]
        db = dstblk.at[k % 2]
        pltpu.async_copy(x_hbm.at[sb.at[pl.ds(0, CHUNK)]], rows0, sem0)
        if k + 1 < NSTAGES:
            stage_idx(k + 1)

        def pair_body(j, _):
            c0 = 2 * j
            c1 = 2 * j + 1
            pltpu.async_copy(x_hbm.at[sb.at[pl.ds(c1 * CHUNK, CHUNK)]],
                             rows1, sem1)
            pltpu.make_async_copy(
                x_hbm.at[sb.at[pl.ds(c0 * CHUNK, CHUNK)]], rows0, sem0).wait()
            pltpu.sync_copy(rows0, acc.at[db.at[c0]], add=True)

            @pl.when(c1 + 1 < IDX_BLK)
            def _():
                pltpu.async_copy(
                    x_hbm.at[sb.at[pl.ds(jnp.minimum(c1 + 1, IDX_BLK - 1)
                                         * CHUNK, CHUNK)]], rows0, sem0)
            pltpu.make_async_copy(
                x_hbm.at[sb.at[pl.ds(c1 * CHUNK, CHUNK)]], rows1, sem1).wait()
            pltpu.sync_copy(rows1, acc.at[db.at[c1]], add=True)
            return 0
        lax.fori_loop(0, IDX_BLK // 2, pair_body, 0)

    @pl.when(wid < TAIL)
    def _():
        off = (TAIL_BASE + wid) * CHUNK
        pltpu.sync_copy(src_hbm.at[pl.ds(off, CHUNK)],
                        srcblk.at[0, pl.ds(0, CHUNK)])
        pltpu.sync_copy(dst_hbm.at[pl.ds(off, CHUNK)],
                        dst1d.at[pl.ds(0, CHUNK)])

        def rcopy(t, _):
            dstblk[0, 0, pl.ds(t * LANES, LANES)] = dst1d[pl.ds(t * LANES,
                                                               LANES)]
            return 0
        lax.fori_loop(0, CHUNK // LANES, rcopy, 0)
        pltpu.async_copy(x_hbm.at[srcblk.at[0, pl.ds(0, CHUNK)]],
                         rows0, sem0).wait()
        pltpu.sync_copy(rows0, acc.at[dstblk.at[0, 0]], add=True)

    plsc.subcore_barrier()

    def wblock(k, _):
        blk = k * NUM_SUBCORES + sid

        @pl.when(blk < NZB)
        def _():
            pltpu.sync_copy(acc.at[pl.ds(blk * ZBLK, ZBLK)],
                            out_hbm.at[cid, pl.ds(blk * ZBLK, ZBLK)])
        return 0
    lax.fori_loop(0, ZB_ITERS, wblock, 0)


@functools.cache
def _sc_agg():
    return pl.kernel(
        _sc_agg_body,
        mesh=plsc.VectorSubcoreMesh(
            core_axis_name="c", subcore_axis_name="s",
            num_cores=NUM_CORES, num_subcores=NUM_SUBCORES),
        out_type=jax.ShapeDtypeStruct((NUM_CORES, N, D), jnp.float32),
        scratch_types=[
            pltpu.VMEM((2, IDX_BLK * CHUNK), jnp.int32),
            pltpu.VMEM((IDX_BLK * CHUNK,), jnp.int32),
            pltpu.VMEM((2, IDX_BLK, CHUNK), jnp.int32),
            pltpu.VMEM((2, CHUNK, D), jnp.float32),
            pltpu.VMEM_SHARED((N, D), jnp.float32),
            pltpu.SemaphoreType.DMA,
            pltpu.SemaphoreType.DMA,
        ],
    )


SPLIT_BLK = E // 8


def _split_body(ei_ref, s_ref, d_ref):
    s_ref[...] = ei_ref[0]
    d_ref[...] = ei_ref[1]


def _split_edges(edge_index):
    return pl.pallas_call(
        _split_body,
        out_shape=[
            jax.ShapeDtypeStruct((E,), jnp.int32),
            jax.ShapeDtypeStruct((E,), jnp.int32),
        ],
    )(edge_index)


ROWS_TC = 1000


def _tc1_body(p_ref, wg_ref, wo_ref, wm_ref, uw_ref, lc_ref, u_ref):
    agg = p_ref[0] + p_ref[1]
    h = jnp.maximum(jnp.dot(agg, wg_ref[...],
                            preferred_element_type=jnp.float32), 0.0)
    od = jnp.dot(h, wo_ref[...], preferred_element_type=jnp.float32)
    q = jnp.dot(od, wm_ref[...], preferred_element_type=jnp.float32)
    col = lax.broadcasted_iota(jnp.int32, (ROWS_TC, D), 1)
    rloc = lax.broadcasted_iota(jnp.int32, (ROWS_TC, D), 0) % NS
    msk = col == rloc
    lc_ref[...] = jnp.concatenate(
        [jnp.sum(jnp.where(msk, q[:, l * D:(l + 1) * D], 0.0), axis=1,
                 keepdims=True) for l in range(LAT)], axis=1)
    u = (jnp.sum(h, axis=1) * (uw_ref[0] / D)
         + jnp.sum(agg, axis=1) * (uw_ref[1] / D))
    u_ref[...] = u.reshape(ROWS_TC, 1)


def _tc1(partials, W_gnn, W_od, W_mat, utility_w):
    grid = N // ROWS_TC
    return pl.pallas_call(
        _tc1_body,
        grid=(grid,),
        in_specs=[
            pl.BlockSpec((NUM_CORES, ROWS_TC, D), lambda i: (0, i, 0)),
            pl.BlockSpec((D, D), lambda i: (0, 0)),
            pl.BlockSpec((D, NS), lambda i: (0, 0)),
            pl.BlockSpec((NS, D * LAT), lambda i: (0, 0)),
            pl.BlockSpec(memory_space=pltpu.SMEM),
        ],
        out_specs=[
            pl.BlockSpec((ROWS_TC, LAT), lambda i: (i, 0)),
            pl.BlockSpec((ROWS_TC, 1), lambda i: (i, 0)),
        ],
        out_shape=[
            jax.ShapeDtypeStruct((N, LAT), jnp.float32),
            jax.ShapeDtypeStruct((N, 1), jnp.float32),
        ],
    )(partials, W_gnn, W_od, W_mat, utility_w)


def _tc2_body(lc_ref, u_ref, be_ref, wb_ref, bb_ref, out_ref):
    grp = lax.broadcasted_iota(jnp.int32, (B, N), 0)
    row = lax.broadcasted_iota(jnp.int32, (B, N), 1)
    gmat = jnp.where(row // NS == grp, 1.0, 0.0)
    lat = jnp.maximum(
        jnp.dot(gmat, lc_ref[...], preferred_element_type=jnp.float32)
        + be_ref[...], 0.0)
    seat = lax.broadcasted_iota(jnp.int32, (N, NS), 1)
    rloc = lax.broadcasted_iota(jnp.int32, (N, NS), 0) % NS
    usel = jnp.where(seat == rloc, u_ref[...], 0.0)
    u2 = jnp.dot(gmat, usel, preferred_element_type=jnp.float32)
    m = jnp.max(u2, axis=1, keepdims=True)
    e = jnp.exp(u2 - m)
    prob = e / jnp.sum(e, axis=1, keepdims=True)
    out_ref[...] = (jnp.dot(lat, wb_ref[...],
                            preferred_element_type=jnp.float32)
                    + bb_ref[...])
    out_ref[:, :NS] = prob


def _tc2(lc, u, b_enc, W_bot, bias_big):
    return pl.pallas_call(
        _tc2_body,
        out_shape=jax.ShapeDtypeStruct((B, NS + N), jnp.float32),
    )(lc, u, b_enc.reshape(1, LAT), W_bot, bias_big)


def kernel(x_od, edge_index, W_gnn, W_od, W_enc, b_enc, W_dec, b_dec, utility_w):
    zsrc = jnp.zeros((ZBLK, D), jnp.float32)
    src, dst = _split_edges(edge_index)
    partials = _sc_agg()(x_od, zsrc, src, dst)
    W_mat = jnp.pad(W_enc.reshape(NS, NS, LAT).transpose(1, 2, 0),
                    ((0, 0), (0, 0), (0, D - NS))).reshape(NS, D * LAT)
    W_bot = jnp.concatenate(
        [jnp.zeros((LAT, NS), jnp.float32), W_dec], axis=1)
    bias_big = jnp.concatenate(
        [jnp.zeros((NS,), jnp.float32), b_dec]).reshape(1, NS + N)
    lc, u = _tc1(partials, W_gnn, W_od, W_mat, utility_w)
    return _tc2(lc, u, b_enc, W_bot, bias_big)
